# baseline (speedup 1.0000x reference)
import functools

import jax
import jax.numpy as jnp
from jax import lax
from jax.experimental import pallas as pl
from jax.experimental.pallas import tpu as pltpu
from jax.experimental.pallas import tpu_sc as plsc

D = 128
D_EDGE = 16
G_SEG = 64
NC, NS = 2, 16
NW = NC * NS
CHUNK = 64
LANES = 16
GRP = 16
SPLIT0 = 0.6


def _edge_proj_body(attr_ref, w1_ref, b1_ref, w2_ref, b2_ref, e1_ref, e2_ref):
    a = attr_ref[...]
    e1_ref[...] = jnp.dot(a, w1_ref[...], preferred_element_type=jnp.float32) + b1_ref[...]
    e2_ref[...] = jnp.dot(a, w2_ref[...], preferred_element_type=jnp.float32) + b2_ref[...]


def _edge_proj(attr_pad, w1, b1, w2, b2):
    e_pad = attr_pad.shape[0]
    be = 2048
    assert e_pad % be == 0
    grid = (e_pad // be,)
    return pl.pallas_call(
        _edge_proj_body,
        grid=grid,
        in_specs=[
            pl.BlockSpec((be, D_EDGE), lambda i: (i, 0)),
            pl.BlockSpec((D_EDGE, D), lambda i: (0, 0)),
            pl.BlockSpec((1, D), lambda i: (0, 0)),
            pl.BlockSpec((D_EDGE, D), lambda i: (0, 0)),
            pl.BlockSpec((1, D), lambda i: (0, 0)),
        ],
        out_specs=[pl.BlockSpec((be, D), lambda i: (i, 0))] * 2,
        out_shape=[jax.ShapeDtypeStruct((e_pad, D), jnp.float32)] * 2,
    )(attr_pad, w1, b1.reshape(1, D), w2, b2.reshape(1, D))


def _mlp_body(x_ref, agg_ref, w1_ref, b1_ref, w2_ref, b2_ref, out_ref):
    h = x_ref[...] + agg_ref[0] + agg_ref[1]
    t = jnp.maximum(jnp.dot(h, w1_ref[...], preferred_element_type=jnp.float32) + b1_ref[...], 0.0)
    out_ref[...] = jnp.maximum(jnp.dot(t, w2_ref[...], preferred_element_type=jnp.float32) + b2_ref[...], 0.0)


def _mlp(x, agg, w1, b1, w2, b2):
    n = x.shape[0]
    nb = 2000
    assert n % nb == 0
    return pl.pallas_call(
        _mlp_body,
        grid=(n // nb,),
        in_specs=[
            pl.BlockSpec((nb, D), lambda i: (i, 0)),
            pl.BlockSpec((2, nb, D), lambda i: (0, i, 0)),
            pl.BlockSpec((D, D), lambda i: (0, 0)),
            pl.BlockSpec((1, D), lambda i: (0, 0)),
            pl.BlockSpec((D, D), lambda i: (0, 0)),
            pl.BlockSpec((1, D), lambda i: (0, 0)),
        ],
        out_specs=pl.BlockSpec((nb, D), lambda i: (i, 0)),
        out_shape=jax.ShapeDtypeStruct((n, D), jnp.float32),
    )(x, agg, w1, b1.reshape(1, D), w2, b2.reshape(1, D))


def _mlp_pool_body(x_ref, agg_ref, seg_ref, w1_ref, b1_ref, w2_ref, b2_ref,
                   out_ref, sums_ref, cnt_ref):
    i = pl.program_id(0)
    nb = x_ref.shape[0]
    h = x_ref[...] + agg_ref[0] + agg_ref[1]
    t = jnp.maximum(jnp.dot(h, w1_ref[...], preferred_element_type=jnp.float32) + b1_ref[...], 0.0)
    o = jnp.maximum(jnp.dot(t, w2_ref[...], preferred_element_type=jnp.float32) + b2_ref[...], 0.0)
    seg = seg_ref[0]
    gid = lax.broadcasted_iota(jnp.int32, (G_SEG, nb), 0)
    oh = (jnp.broadcast_to(seg, (G_SEG, nb)) == gid).astype(jnp.float32)
    ps = jnp.dot(oh, o, preferred_element_type=jnp.float32)
    pc = jnp.broadcast_to(jnp.sum(oh, axis=1, keepdims=True), (G_SEG, D))

    @pl.when(i == 0)
    def _():
        sums_ref[...] = ps
        cnt_ref[...] = pc

    @pl.when(i > 0)
    def _():
        sums_ref[...] += ps
        cnt_ref[...] += pc

    @pl.when(i == pl.num_programs(0) - 1)
    def _():
        out_ref[...] = sums_ref[...] / jnp.maximum(cnt_ref[...], 1.0)


def _mlp_pool(x, agg, seg3, w1, b1, w2, b2):
    n = x.shape[0]
    nb = 2000
    assert n % nb == 0
    return pl.pallas_call(
        _mlp_pool_body,
        grid=(n // nb,),
        in_specs=[
            pl.BlockSpec((nb, D), lambda i: (i, 0)),
            pl.BlockSpec((2, nb, D), lambda i: (0, i, 0)),
            pl.BlockSpec((1, 1, nb), lambda i: (i, 0, 0)),
            pl.BlockSpec((D, D), lambda i: (0, 0)),
            pl.BlockSpec((1, D), lambda i: (0, 0)),
            pl.BlockSpec((D, D), lambda i: (0, 0)),
            pl.BlockSpec((1, D), lambda i: (0, 0)),
        ],
        out_specs=pl.BlockSpec((G_SEG, D), lambda i: (0, 0)),
        out_shape=jax.ShapeDtypeStruct((G_SEG, D), jnp.float32),
        scratch_shapes=[
            pltpu.VMEM((G_SEG, D), jnp.float32),
            pltpu.VMEM((G_SEG, D), jnp.float32),
        ],
    )(x, agg, seg3, w1, b1.reshape(1, D), w2, b2.reshape(1, D))


def _sc_aggregate(x, e, idx3, n_pad, cpt0, cpt1):
    assert cpt0 % (2 * GRP) == 0 and cpt1 % (2 * GRP) == 0
    rows_per_tile = n_pad // NS
    zc = rows_per_tile // CHUNK
    assert rows_per_tile % CHUNK == 0
    mesh = plsc.VectorSubcoreMesh(core_axis_name="c", subcore_axis_name="s",
                                  num_cores=NC, num_subcores=NS)

    @functools.partial(
        pl.kernel,
        out_type=jax.ShapeDtypeStruct((NC, n_pad, D), jnp.float32),
        mesh=mesh,
        scratch_types=[
            pltpu.VMEM((2 * GRP, CHUNK), jnp.int32),
            pltpu.VMEM((CHUNK, D), jnp.float32),
            pltpu.VMEM((CHUNK, D), jnp.float32),
            pltpu.VMEM((CHUNK, D), jnp.float32),
            pltpu.VMEM((CHUNK, D), jnp.float32),
            pltpu.VMEM_SHARED((n_pad, D), jnp.float32),
            pltpu.SemaphoreType.DMA,
            pltpu.SemaphoreType.DMA,
            pltpu.SemaphoreType.DMA,
            pltpu.SemaphoreType.DMA,
            pltpu.SemaphoreType.DMA,
            pltpu.SemaphoreType.DMA,
        ],
    )
    def body(x_hbm, e_hbm, idx_hbm, out_hbm,
             idx_v, xra, xrb, era, erb, accum,
             sga, sgb, sea, seb, ssa, ssb):
        cid = lax.axis_index("c")
        sid = lax.axis_index("s")
        base = jnp.where(cid == 0, sid * cpt0, NS * cpt0 + sid * cpt1)
        npairs_c = jnp.where(cid == 0, cpt0 // 2, cpt1 // 2)
        gbase = base // GRP

        @plsc.parallel_loop(0, CHUNK, 1, unroll=4)
        def zrow(r):
            for c in range(D // LANES):
                xra[r, pl.ds(c * LANES, LANES)] = jnp.zeros((LANES,), jnp.float32)

        def zcopy(i, _):
            pltpu.sync_copy(xra, accum.at[pl.ds(sid * rows_per_tile + i * CHUNK, CHUNK)])
            return 0
        lax.fori_loop(0, zc, zcopy, 0)
        plsc.subcore_barrier()

        def compute(xr, er):
            @plsc.parallel_loop(0, CHUNK, 1, unroll=4)
            def crow(rr):
                for c in range(D // LANES):
                    s = pl.ds(c * LANES, LANES)
                    xr[rr, s] = jnp.maximum(xr[rr, s] + er[rr, s], 0.0)

        def issue_ge(j, r, xr, er, sg, se):
            dg = pltpu.async_copy(x_hbm.at[idx_v.at[r]], xr, sg)
            de = pltpu.async_copy(
                e_hbm.at[pl.ds((base + j) * CHUNK, CHUNK)], er, se)
            return dg, de

        def drain_ge(xr, er, sg, se):
            pltpu.make_async_copy(x_hbm.at[idx_v.at[0]], xr, sg).wait()
            pltpu.make_async_copy(e_hbm.at[pl.ds(0, CHUNK)], er, se).wait()

        def issue_sc(r, xr, ss):
            pltpu.async_copy(xr, accum.at[idx_v.at[GRP + r]], ss, add=True)

        def drain_sc(xr, ss):
            pltpu.make_async_copy(xr, accum.at[idx_v.at[GRP]], ss).wait()

        pltpu.sync_copy(idx_hbm.at[gbase], idx_v)
        issue_ge(0, 0, xra, era, sga, sea)

        ppg = GRP // 2

        def pair_body(i, _):
            j0 = 2 * i
            j1 = j0 + 1
            r0 = lax.rem(j0, GRP)
            boundary = lax.rem(i, ppg) == ppg - 1

            @pl.when(jnp.logical_and(i > 0, lax.rem(i, ppg) != 0))
            def _():
                drain_sc(xrb, ssb)
            db = issue_ge(j1, r0 + 1, xrb, erb, sgb, seb)
            drain_ge(xra, era, sga, sea)
            compute(xra, era)
            issue_sc(r0, xra, ssa)
            db[0].wait()
            db[1].wait()
            compute(xrb, erb)
            issue_sc(r0 + 1, xrb, ssb)
            drain_sc(xra, ssa)

            @pl.when(boundary)
            def _():
                drain_sc(xrb, ssb)

            @pl.when(jnp.logical_and(boundary, i + 1 < npairs_c))
            def _():
                pltpu.sync_copy(idx_hbm.at[gbase + (i + 1) // ppg], idx_v)

            @pl.when(i + 1 < npairs_c)
            def _():
                issue_ge(j0 + 2, lax.rem(j0 + 2, GRP), xra, era, sga, sea)
            return 0
        lax.fori_loop(0, npairs_c, pair_body, 0)
        plsc.subcore_barrier()

        def wcopy(i, _):
            sl = pl.ds(sid * rows_per_tile + i * CHUNK, CHUNK)
            pltpu.sync_copy(accum.at[sl], xra)
            pltpu.sync_copy(xra, out_hbm.at[cid, sl])
            return 0
        lax.fori_loop(0, zc, wcopy, 0)

    return body(x, e, idx3)


def kernel(x, edge_index, edge_attr, batch,
           lin1_W, lin1_b, mlp1_W1, mlp1_b1, mlp1_W2, mlp1_b2,
           lin2_W, lin2_b, mlp2_W1, mlp2_b1, mlp2_W2, mlp2_b2):
    n = x.shape[0]
    e_num = edge_index.shape[1]
    cpt = -(-e_num // (NW * CHUNK))
    cpt = -(-cpt // (2 * GRP)) * (2 * GRP)
    total_pair = 2 * cpt
    cpt0 = int(round(total_pair * SPLIT0 / (2 * GRP))) * (2 * GRP)
    cpt0 = min(max(cpt0, 2 * GRP), total_pair - 2 * GRP)
    cpt1 = total_pair - cpt0
    e_pad = NS * total_pair * CHUNK
    n_pad = -(-(n + 1) // (NS * CHUNK)) * (NS * CHUNK)
    trash = n

    src = jnp.pad(edge_index[0], (0, e_pad - e_num))
    dst = jnp.pad(edge_index[1], (0, e_pad - e_num), constant_values=trash)
    attr_pad = jnp.pad(edge_attr, ((0, e_pad - e_num), (0, 0)))
    src3 = src.reshape(-1, GRP, CHUNK)
    dst3 = dst.reshape(-1, GRP, CHUNK)
    idx3 = jnp.concatenate([src3, dst3], axis=1)

    e1, e2 = _edge_proj(attr_pad, lin1_W, lin1_b, lin2_W, lin2_b)

    agg1 = _sc_aggregate(x, e1, idx3, n_pad, cpt0, cpt1)
    h1 = _mlp(x, agg1, mlp1_W1, mlp1_b1, mlp1_W2, mlp1_b2)

    agg2 = _sc_aggregate(h1, e2, idx3, n_pad, cpt0, cpt1)
    seg3 = batch.reshape(n // 2000, 1, 2000)
    return _mlp_pool(h1, agg2, seg3, mlp2_W1, mlp2_b1, mlp2_W2, mlp2_b2)


# speedup vs baseline: 1.0790x; 1.0790x over previous
import functools

import jax
import jax.numpy as jnp
from jax import lax
from jax.experimental import pallas as pl
from jax.experimental.pallas import tpu as pltpu
from jax.experimental.pallas import tpu_sc as plsc

D = 128
D_EDGE = 16
G_SEG = 64
NC, NS = 2, 16
NW = NC * NS
CHUNK = 64
LANES = 16
GRP = 16
SPLIT0 = 0.7


def _edge_proj_body(attr_ref, w1_ref, b1_ref, w2_ref, b2_ref, e1_ref, e2_ref):
    a = attr_ref[...]
    e1_ref[...] = jnp.dot(a, w1_ref[...], preferred_element_type=jnp.float32) + b1_ref[...]
    e2_ref[...] = jnp.dot(a, w2_ref[...], preferred_element_type=jnp.float32) + b2_ref[...]


def _edge_proj(attr_pad, w1, b1, w2, b2):
    e_pad = attr_pad.shape[0]
    be = 2048
    assert e_pad % be == 0
    grid = (e_pad // be,)
    return pl.pallas_call(
        _edge_proj_body,
        grid=grid,
        in_specs=[
            pl.BlockSpec((be, D_EDGE), lambda i: (i, 0)),
            pl.BlockSpec((D_EDGE, D), lambda i: (0, 0)),
            pl.BlockSpec((1, D), lambda i: (0, 0)),
            pl.BlockSpec((D_EDGE, D), lambda i: (0, 0)),
            pl.BlockSpec((1, D), lambda i: (0, 0)),
        ],
        out_specs=[pl.BlockSpec((be, D), lambda i: (i, 0))] * 2,
        out_shape=[jax.ShapeDtypeStruct((e_pad, D), jnp.float32)] * 2,
    )(attr_pad, w1, b1.reshape(1, D), w2, b2.reshape(1, D))


def _mlp_body(x_ref, agg_ref, w1_ref, b1_ref, w2_ref, b2_ref, out_ref):
    h = x_ref[...] + agg_ref[0] + agg_ref[1]
    t = jnp.maximum(jnp.dot(h, w1_ref[...], preferred_element_type=jnp.float32) + b1_ref[...], 0.0)
    out_ref[...] = jnp.maximum(jnp.dot(t, w2_ref[...], preferred_element_type=jnp.float32) + b2_ref[...], 0.0)


def _mlp(x, agg, w1, b1, w2, b2):
    n = x.shape[0]
    nb = 2000
    assert n % nb == 0
    return pl.pallas_call(
        _mlp_body,
        grid=(n // nb,),
        in_specs=[
            pl.BlockSpec((nb, D), lambda i: (i, 0)),
            pl.BlockSpec((2, nb, D), lambda i: (0, i, 0)),
            pl.BlockSpec((D, D), lambda i: (0, 0)),
            pl.BlockSpec((1, D), lambda i: (0, 0)),
            pl.BlockSpec((D, D), lambda i: (0, 0)),
            pl.BlockSpec((1, D), lambda i: (0, 0)),
        ],
        out_specs=pl.BlockSpec((nb, D), lambda i: (i, 0)),
        out_shape=jax.ShapeDtypeStruct((n, D), jnp.float32),
    )(x, agg, w1, b1.reshape(1, D), w2, b2.reshape(1, D))


def _mlp_pool_body(x_ref, agg_ref, seg_ref, w1_ref, b1_ref, w2_ref, b2_ref,
                   out_ref, sums_ref, cnt_ref):
    i = pl.program_id(0)
    nb = x_ref.shape[0]
    h = x_ref[...] + agg_ref[0] + agg_ref[1]
    t = jnp.maximum(jnp.dot(h, w1_ref[...], preferred_element_type=jnp.float32) + b1_ref[...], 0.0)
    o = jnp.maximum(jnp.dot(t, w2_ref[...], preferred_element_type=jnp.float32) + b2_ref[...], 0.0)
    seg = seg_ref[0]
    gid = lax.broadcasted_iota(jnp.int32, (G_SEG, nb), 0)
    oh = (jnp.broadcast_to(seg, (G_SEG, nb)) == gid).astype(jnp.float32)
    ps = jnp.dot(oh, o, preferred_element_type=jnp.float32)
    pc = jnp.broadcast_to(jnp.sum(oh, axis=1, keepdims=True), (G_SEG, D))

    @pl.when(i == 0)
    def _():
        sums_ref[...] = ps
        cnt_ref[...] = pc

    @pl.when(i > 0)
    def _():
        sums_ref[...] += ps
        cnt_ref[...] += pc

    @pl.when(i == pl.num_programs(0) - 1)
    def _():
        out_ref[...] = sums_ref[...] / jnp.maximum(cnt_ref[...], 1.0)


def _mlp_pool(x, agg, seg3, w1, b1, w2, b2):
    n = x.shape[0]
    nb = 2000
    assert n % nb == 0
    return pl.pallas_call(
        _mlp_pool_body,
        grid=(n // nb,),
        in_specs=[
            pl.BlockSpec((nb, D), lambda i: (i, 0)),
            pl.BlockSpec((2, nb, D), lambda i: (0, i, 0)),
            pl.BlockSpec((1, 1, nb), lambda i: (i, 0, 0)),
            pl.BlockSpec((D, D), lambda i: (0, 0)),
            pl.BlockSpec((1, D), lambda i: (0, 0)),
            pl.BlockSpec((D, D), lambda i: (0, 0)),
            pl.BlockSpec((1, D), lambda i: (0, 0)),
        ],
        out_specs=pl.BlockSpec((G_SEG, D), lambda i: (0, 0)),
        out_shape=jax.ShapeDtypeStruct((G_SEG, D), jnp.float32),
        scratch_shapes=[
            pltpu.VMEM((G_SEG, D), jnp.float32),
            pltpu.VMEM((G_SEG, D), jnp.float32),
        ],
    )(x, agg, seg3, w1, b1.reshape(1, D), w2, b2.reshape(1, D))


def _sc_aggregate(x, e, idx3, n_pad, cpt0, cpt1):
    assert cpt0 % (2 * GRP) == 0 and cpt1 % (2 * GRP) == 0
    rows_per_tile = n_pad // NS
    zc = rows_per_tile // CHUNK
    assert rows_per_tile % CHUNK == 0
    mesh = plsc.VectorSubcoreMesh(core_axis_name="c", subcore_axis_name="s",
                                  num_cores=NC, num_subcores=NS)

    @functools.partial(
        pl.kernel,
        out_type=jax.ShapeDtypeStruct((NC, n_pad, D), jnp.float32),
        mesh=mesh,
        scratch_types=[
            pltpu.VMEM((2 * GRP, CHUNK), jnp.int32),
            pltpu.VMEM((CHUNK, D), jnp.float32),
            pltpu.VMEM((CHUNK, D), jnp.float32),
            pltpu.VMEM((CHUNK, D), jnp.float32),
            pltpu.VMEM((CHUNK, D), jnp.float32),
            pltpu.VMEM_SHARED((n_pad, D), jnp.float32),
            pltpu.SemaphoreType.DMA,
            pltpu.SemaphoreType.DMA,
            pltpu.SemaphoreType.DMA,
            pltpu.SemaphoreType.DMA,
            pltpu.SemaphoreType.DMA,
            pltpu.SemaphoreType.DMA,
        ],
    )
    def body(x_hbm, e_hbm, idx_hbm, out_hbm,
             idx_v, xra, xrb, era, erb, accum,
             sga, sgb, sea, seb, ssa, ssb):
        cid = lax.axis_index("c")
        sid = lax.axis_index("s")
        base = jnp.where(cid == 0, sid * cpt0, NS * cpt0 + sid * cpt1)
        npairs_c = jnp.where(cid == 0, cpt0 // 2, cpt1 // 2)
        gbase = base // GRP

        @plsc.parallel_loop(0, CHUNK, 1, unroll=4)
        def zrow(r):
            for c in range(D // LANES):
                xra[r, pl.ds(c * LANES, LANES)] = jnp.zeros((LANES,), jnp.float32)

        def zcopy(i, _):
            pltpu.sync_copy(xra, accum.at[pl.ds(sid * rows_per_tile + i * CHUNK, CHUNK)])
            return 0
        lax.fori_loop(0, zc, zcopy, 0)
        plsc.subcore_barrier()

        def compute(xr, er):
            @plsc.parallel_loop(0, CHUNK, 1, unroll=4)
            def crow(rr):
                for c in range(D // LANES):
                    s = pl.ds(c * LANES, LANES)
                    xr[rr, s] = jnp.maximum(xr[rr, s] + er[rr, s], 0.0)

        def issue_ge(j, r, xr, er, sg, se):
            dg = pltpu.async_copy(x_hbm.at[idx_v.at[r]], xr, sg)
            de = pltpu.async_copy(
                e_hbm.at[pl.ds((base + j) * CHUNK, CHUNK)], er, se)
            return dg, de

        def drain_ge(xr, er, sg, se):
            pltpu.make_async_copy(x_hbm.at[idx_v.at[0]], xr, sg).wait()
            pltpu.make_async_copy(e_hbm.at[pl.ds(0, CHUNK)], er, se).wait()

        def issue_sc(r, xr, ss):
            pltpu.async_copy(xr, accum.at[idx_v.at[GRP + r]], ss, add=True)

        def drain_sc(xr, ss):
            pltpu.make_async_copy(xr, accum.at[idx_v.at[GRP]], ss).wait()

        pltpu.sync_copy(idx_hbm.at[gbase], idx_v)
        issue_ge(0, 0, xra, era, sga, sea)

        ppg = GRP // 2

        def pair_body(i, _):
            j0 = 2 * i
            j1 = j0 + 1
            r0 = lax.rem(j0, GRP)
            boundary = lax.rem(i, ppg) == ppg - 1

            @pl.when(jnp.logical_and(i > 0, lax.rem(i, ppg) != 0))
            def _():
                drain_sc(xrb, ssb)
            db = issue_ge(j1, r0 + 1, xrb, erb, sgb, seb)
            drain_ge(xra, era, sga, sea)
            compute(xra, era)
            issue_sc(r0, xra, ssa)
            db[0].wait()
            db[1].wait()
            compute(xrb, erb)
            issue_sc(r0 + 1, xrb, ssb)
            drain_sc(xra, ssa)

            @pl.when(boundary)
            def _():
                drain_sc(xrb, ssb)

            @pl.when(jnp.logical_and(boundary, i + 1 < npairs_c))
            def _():
                pltpu.sync_copy(idx_hbm.at[gbase + (i + 1) // ppg], idx_v)

            @pl.when(i + 1 < npairs_c)
            def _():
                issue_ge(j0 + 2, lax.rem(j0 + 2, GRP), xra, era, sga, sea)
            return 0
        lax.fori_loop(0, npairs_c, pair_body, 0)
        plsc.subcore_barrier()

        def wcopy(i, _):
            sl = pl.ds(sid * rows_per_tile + i * CHUNK, CHUNK)
            pltpu.sync_copy(accum.at[sl], xra)
            pltpu.sync_copy(xra, out_hbm.at[cid, sl])
            return 0
        lax.fori_loop(0, zc, wcopy, 0)

    return body(x, e, idx3)


def kernel(x, edge_index, edge_attr, batch,
           lin1_W, lin1_b, mlp1_W1, mlp1_b1, mlp1_W2, mlp1_b2,
           lin2_W, lin2_b, mlp2_W1, mlp2_b1, mlp2_W2, mlp2_b2):
    n = x.shape[0]
    e_num = edge_index.shape[1]
    cpt = -(-e_num // (NW * CHUNK))
    cpt = -(-cpt // (2 * GRP)) * (2 * GRP)
    total_pair = 2 * cpt
    cpt0 = int(round(total_pair * SPLIT0 / (2 * GRP))) * (2 * GRP)
    cpt0 = min(max(cpt0, 2 * GRP), total_pair - 2 * GRP)
    cpt1 = total_pair - cpt0
    e_pad = NS * total_pair * CHUNK
    n_pad = -(-(n + 1) // (NS * CHUNK)) * (NS * CHUNK)
    trash = n

    src = jnp.pad(edge_index[0], (0, e_pad - e_num))
    dst = jnp.pad(edge_index[1], (0, e_pad - e_num), constant_values=trash)
    attr_pad = jnp.pad(edge_attr, ((0, e_pad - e_num), (0, 0)))
    src3 = src.reshape(-1, GRP, CHUNK)
    dst3 = dst.reshape(-1, GRP, CHUNK)
    idx3 = jnp.concatenate([src3, dst3], axis=1)

    e1, e2 = _edge_proj(attr_pad, lin1_W, lin1_b, lin2_W, lin2_b)

    agg1 = _sc_aggregate(x, e1, idx3, n_pad, cpt0, cpt1)
    h1 = _mlp(x, agg1, mlp1_W1, mlp1_b1, mlp1_W2, mlp1_b2)

    agg2 = _sc_aggregate(h1, e2, idx3, n_pad, cpt0, cpt1)
    seg3 = batch.reshape(n // 2000, 1, 2000)
    return _mlp_pool(h1, agg2, seg3, mlp2_W1, mlp2_b1, mlp2_W2, mlp2_b2)
